# baseline (speedup 1.0000x reference)
import functools

import jax
import jax.numpy as jnp
from jax import lax
from jax.experimental import pallas as pl
from jax.experimental.pallas import tpu as pltpu
from jax.experimental.pallas import tpu_sc as plsc

_VOCAB = 100000
_D = 128
_NP = 20
_B = 4
_T = 2048

_NC = 2
_NS = 16
_NW = _NC * _NS
_BT = _B * _T
_PER_W = _BT // _NW
_IDX_CHUNK = 128
_N_CHUNKS = _PER_W // _IDX_CHUNK

_mesh = plsc.VectorSubcoreMesh(
    core_axis_name="c", subcore_axis_name="s", num_cores=_NC, num_subcores=_NS
)


@functools.partial(
    pl.kernel,
    out_type=jax.ShapeDtypeStruct((_B, _NP + _T, _D), jnp.float32),
    mesh=_mesh,
    scratch_types=[
        pltpu.VMEM((_N_CHUNKS, _IDX_CHUNK), jnp.int32),
        pltpu.VMEM((_PER_W, _D), jnp.float32),
        pltpu.VMEM((_NP, _D), jnp.float32),
        pltpu.SemaphoreType.DMA,
        pltpu.SemaphoreType.DMA,
    ],
    compiler_params=pltpu.CompilerParams(use_tc_tiling_on_sc=False),
)
def _soft_prompt_embed(ids_hbm, table_hbm, prompt_hbm, out_hbm,
                       idx_v, rows_v, prm_v, gsem, psem):
    wid = lax.axis_index("s") * _NC + lax.axis_index("c")
    base = wid * _PER_W
    batch = base // _T
    col = base - batch * _T

    for j in range(_N_CHUNKS):
        pltpu.sync_copy(
            ids_hbm.at[batch, pl.ds(col + j * _IDX_CHUNK, _IDX_CHUNK)],
            idx_v.at[j],
        )
    for j in range(_N_CHUNKS):
        pltpu.async_copy(
            table_hbm.at[idx_v.at[j]],
            rows_v.at[pl.ds(j * _IDX_CHUNK, _IDX_CHUNK)],
            gsem,
        )

    @pl.when(wid < _B)
    def _():
        copy = pltpu.async_copy(prompt_hbm, prm_v, psem)
        copy.wait()
        pltpu.sync_copy(prm_v, out_hbm.at[wid, pl.ds(0, _NP)])

    for j in range(_N_CHUNKS):
        pltpu.make_async_copy(
            table_hbm.at[idx_v.at[j]],
            rows_v.at[pl.ds(j * _IDX_CHUNK, _IDX_CHUNK)],
            gsem,
        ).wait()
    pltpu.sync_copy(rows_v, out_hbm.at[batch, pl.ds(_NP + col, _PER_W)])


def kernel(input_ids, table, soft_prompt):
    return _soft_prompt_embed(input_ids.astype(jnp.int32), table, soft_prompt)


# speedup vs baseline: 1.4472x; 1.4472x over previous
import functools

import jax
import jax.numpy as jnp
from jax import lax
from jax.experimental import pallas as pl
from jax.experimental.pallas import tpu as pltpu
from jax.experimental.pallas import tpu_sc as plsc

_VOCAB = 100000
_D = 128
_NP = 20
_B = 4
_T = 2048

_NC = 2
_NS = 16
_NW = _NC * _NS
_TPW = _T // _NW
_ROWS_W = _TPW * _B

_mesh = plsc.VectorSubcoreMesh(
    core_axis_name="c", subcore_axis_name="s", num_cores=_NC, num_subcores=_NS
)


@functools.partial(
    pl.kernel,
    out_type=jax.ShapeDtypeStruct((_NP + _T, _B, _D), jnp.float32),
    mesh=_mesh,
    scratch_types=[
        pltpu.VMEM((_B, _TPW), jnp.int32),
        pltpu.VMEM((_ROWS_W, _D), jnp.float32),
        pltpu.VMEM((1, _D), jnp.float32),
        pltpu.SemaphoreType.DMA,
        pltpu.SemaphoreType.DMA,
        pltpu.SemaphoreType.DMA,
    ],
    compiler_params=pltpu.CompilerParams(use_tc_tiling_on_sc=False),
)
def _soft_prompt_embed(ids_hbm, table_hbm, prompt_hbm, out_hbm,
                       idx_v, rows_v, prm_v, gsem, psem, ssem):
    wid = lax.axis_index("s") * _NC + lax.axis_index("c")
    t0 = wid * _TPW
    blk = t0 // 128
    col = t0 - blk * 128

    pltpu.sync_copy(ids_hbm.at[blk, :, pl.ds(col, _TPW)], idx_v)
    for b in range(_B):
        pltpu.async_copy(
            table_hbm.at[idx_v.at[b]],
            rows_v.at[pl.ds(b * _TPW, _TPW)],
            gsem,
        )

    @pl.when(wid < _NP)
    def _():
        pltpu.async_copy(prompt_hbm.at[pl.ds(wid, 1)], prm_v, psem).wait()
        for b in range(_B):
            pltpu.async_copy(prm_v, out_hbm.at[pl.ds(wid, 1), b], psem)
        for b in range(_B):
            pltpu.make_async_copy(prm_v, out_hbm.at[pl.ds(wid, 1), b], psem).wait()

    for b in range(_B):
        pltpu.make_async_copy(
            table_hbm.at[idx_v.at[b]],
            rows_v.at[pl.ds(b * _TPW, _TPW)],
            gsem,
        ).wait()
        pltpu.async_copy(
            rows_v.at[pl.ds(b * _TPW, _TPW)],
            out_hbm.at[pl.ds(_NP + t0, _TPW), b],
            ssem,
        )
    for b in range(_B):
        pltpu.make_async_copy(
            rows_v.at[pl.ds(b * _TPW, _TPW)],
            out_hbm.at[pl.ds(_NP + t0, _TPW), b],
            ssem,
        ).wait()


def kernel(input_ids, table, soft_prompt):
    ids_blocks = input_ids.astype(jnp.int32).reshape(_B, _T // 128, 128)
    ids_blocks = jnp.transpose(ids_blocks, (1, 0, 2))
    out = _soft_prompt_embed(ids_blocks, table, soft_prompt)
    return jnp.transpose(out, (1, 0, 2))
